# baseline (speedup 1.0000x reference)
import functools

import jax
import jax.numpy as jnp
from jax import lax
from jax.experimental import pallas as pl
from jax.experimental.pallas import tpu as pltpu
from jax.experimental.pallas import tpu_sc as plsc

N = 10000
E = 320000
D = 128
H = 64
C = 64
NITER = 10

NCORE = 2
NSUB = 16
HALF = C // NCORE
NPAD = 10240
RPT = NPAD // NSUB
NRSUB = RPT // 128
EPT = 20480
EPAD = EPT * NSUB
NCHUNK = EPT // 128


def _mlp_body(x_ref, w1_ref, b1_ref, w2_ref, b2_ref, o_ref):
    x = x_ref[...]
    h1 = lax.dot_general(x, w1_ref[...], (((1,), (1,)), ((), ())),
                         preferred_element_type=jnp.float32)
    h1 = jnp.maximum(h1 + b1_ref[...], 0.0)
    h2 = lax.dot_general(h1, w2_ref[...], (((1,), (1,)), ((), ())),
                         preferred_element_type=jnp.float32)
    o_ref[...] = h2 + b2_ref[...]


def _mlp(x, W1, b1, W2, b2):
    blk = 1000
    grid = N // blk
    return pl.pallas_call(
        _mlp_body,
        grid=(grid,),
        in_specs=[
            pl.BlockSpec((blk, D), lambda i: (i, 0)),
            pl.BlockSpec((H, D), lambda i: (0, 0)),
            pl.BlockSpec((1, H), lambda i: (0, 0)),
            pl.BlockSpec((C, H), lambda i: (0, 0)),
            pl.BlockSpec((1, C), lambda i: (0, 0)),
        ],
        out_specs=pl.BlockSpec((blk, C), lambda i: (i, 0)),
        out_shape=jax.ShapeDtypeStruct((N, C), jnp.float32),
    )(x, W1, b1.reshape(1, H), W2, b2.reshape(1, C))


def _rsqrt16(x):
    s = (x + 1.0) * 0.5
    for _ in range(8):
        s = 0.5 * (s + x / s)
    return 1.0 / s


CHW = 512
NSUP = EPT // CHW


def _prop_body(eidx, h_hbm, outs_hbm, u_hbm,
               src_v, dst_v, gbuf, ones_v, degbuf,
               dinv1_v, deg_sh, acc_sh, semg, sems):
    core = lax.axis_index("c")
    wid = lax.axis_index("s")
    hbuf = gbuf.at[pl.ds(0, 128)]
    ubuf = gbuf.at[pl.ds(128, 128)]
    accbuf = gbuf.at[pl.ds(256, 128)]
    rbase = wid * RPT
    ubase = core * NPAD

    pltpu.sync_copy(eidx.at[0, pl.ds(wid * NSUP, NSUP)], src_v)
    pltpu.sync_copy(eidx.at[1, pl.ds(wid * NSUP, NSUP)], dst_v)

    def _off_body(k, _):
        r = k // (CHW // 16)
        c = (k % (CHW // 16)) * 16
        src_v[r, pl.ds(c, 16)] = src_v[r, pl.ds(c, 16)] + ubase
        return 0
    lax.fori_loop(0, EPT // 16, _off_body, 0)

    def _ones_body(k, _):
        ones_v[k, :] = jnp.full((16,), 1.0, jnp.float32)
        return 0
    lax.fori_loop(0, CHW, _ones_body, 0)

    def _dinit_body(s, _):
        pltpu.sync_copy(ones_v.at[pl.ds(0, 128)],
                        deg_sh.at[pl.ds(rbase + s * 128, 128)])
        return 0
    lax.fori_loop(0, NRSUB, _dinit_body, 0)
    plsc.subcore_barrier()

    def _deg_body(j, _):
        pltpu.sync_copy(ones_v, deg_sh.at[dst_v.at[j]], add=True)
        return 0
    lax.fori_loop(0, NSUP, _deg_body, 0)
    plsc.subcore_barrier()

    def _dinv_sub(s, _):
        pltpu.sync_copy(deg_sh.at[pl.ds(rbase + s * 128, 128)], degbuf)

        def _row(i, _):
            dinv1_v[s * 128 + i, :] = _rsqrt16(degbuf[i, :])
            return 0
        lax.fori_loop(0, 128, _row, 0)
        return 0
    lax.fori_loop(0, NRSUB, _dinv_sub, 0)

    def _u0_sub(s, _):
        pltpu.sync_copy(h_hbm.at[pl.ds(ubase + rbase + s * 128, 128)], hbuf)

        def _row(i, _):
            dv = dinv1_v[s * 128 + i, :]
            for p in range(HALF // 16):
                ubuf[i, pl.ds(p * 16, 16)] = hbuf[i, pl.ds(p * 16, 16)] * dv
            return 0
        lax.fori_loop(0, 128, _row, 0)
        pltpu.sync_copy(ubuf, u_hbm.at[pl.ds(ubase + rbase + s * 128, 128)])
        pltpu.sync_copy(ubuf, acc_sh.at[pl.ds(rbase + s * 128, 128)])
        return 0
    lax.fori_loop(0, NRSUB, _u0_sub, 0)
    plsc.subcore_barrier()

    def _round(t, _):
        pltpu.async_copy(u_hbm.at[src_v.at[0]],
                         gbuf.at[pl.ds(0, CHW)], semg.at[0])

        def _grp(g, _):
            for b in range(2):
                s = g * 2 + b
                bb = (b + 1) % 2
                slot = pl.ds(b * CHW, CHW)
                oth = pl.ds(bb * CHW, CHW)
                pltpu.make_async_copy(
                    u_hbm.at[src_v.at[s]], gbuf.at[slot], semg.at[b]).wait()
                pltpu.async_copy(gbuf.at[slot], acc_sh.at[dst_v.at[s]],
                                 sems.at[b], add=True)

                @pl.when(s + 1 < NSUP)
                def _():
                    @pl.when(s >= 1)
                    def _():
                        pltpu.make_async_copy(
                            gbuf.at[oth], acc_sh.at[dst_v.at[s - 1]],
                            sems.at[bb]).wait()
                    pltpu.async_copy(u_hbm.at[src_v.at[s + 1]],
                                     gbuf.at[oth], semg.at[bb])
            return 0
        lax.fori_loop(0, NSUP // 2, _grp, 0)
        for b in range(2):
            sl = NSUP - 2 + b
            pltpu.make_async_copy(gbuf.at[pl.ds(b * CHW, CHW)],
                                  acc_sh.at[dst_v.at[sl]], sems.at[b]).wait()
        plsc.subcore_barrier()

        def _scale_sub(s, _):
            row0 = rbase + s * 128
            pltpu.sync_copy(acc_sh.at[pl.ds(row0, 128)], accbuf)

            def _row(i, _):
                d1 = dinv1_v[s * 128 + i, :]
                d2 = d1 * d1
                for p in range(HALF // 16):
                    a = accbuf[i, pl.ds(p * 16, 16)]
                    hbuf[i, pl.ds(p * 16, 16)] = a * d1
                    ubuf[i, pl.ds(p * 16, 16)] = a * d2
                return 0
            lax.fori_loop(0, 128, _row, 0)
            orow = t * (NCORE * NPAD) + ubase + row0
            pltpu.sync_copy(hbuf, outs_hbm.at[pl.ds(orow, 128)])
            pltpu.sync_copy(ubuf, u_hbm.at[pl.ds(ubase + row0, 128)])
            pltpu.sync_copy(ubuf, acc_sh.at[pl.ds(row0, 128)])
            return 0
        lax.fori_loop(0, NRSUB, _scale_sub, 0)
        plsc.subcore_barrier()
        return 0
    lax.fori_loop(0, NITER, _round, 0)


def _propagate(eidx_pad, h_split):
    mesh = plsc.VectorSubcoreMesh(core_axis_name="c", subcore_axis_name="s",
                                  num_cores=NCORE, num_subcores=NSUB)
    f = pl.kernel(
        _prop_body,
        out_type=(
            jax.ShapeDtypeStruct((NITER * NCORE * NPAD, HALF), jnp.float32),
            jax.ShapeDtypeStruct((NCORE * NPAD, HALF), jnp.float32),
        ),
        mesh=mesh,
        scratch_types=[
            pltpu.VMEM((NSUP, CHW), jnp.int32),
            pltpu.VMEM((NSUP, CHW), jnp.int32),
            pltpu.VMEM((2 * CHW, HALF), jnp.float32),
            pltpu.VMEM((CHW, 16), jnp.float32),
            pltpu.VMEM((128, 16), jnp.float32),
            pltpu.VMEM((RPT, 16), jnp.float32),
            pltpu.VMEM_SHARED((NPAD, 16), jnp.float32),
            pltpu.VMEM_SHARED((NPAD, HALF), jnp.float32),
            pltpu.SemaphoreType.DMA((2,)),
            pltpu.SemaphoreType.DMA((2,)),
        ],
        compiler_params=pltpu.CompilerParams(use_tc_tiling_on_sc=False),
    )
    return f(eidx_pad, h_split)


def _halt_body(o_ref, wh_ref, bh_ref, lo_ref, p_ref):
    wh = wh_ref[...]
    rows = []
    for t in range(NITER):
        lt = jnp.sum(o_ref[t] * wh, axis=1)[None, :] + bh_ref[...]
        rows.append(jnp.clip(lt, -10.0, 10.0))
    logits = jnp.concatenate(rows, axis=0)
    lam = 1.0 / (1.0 + jnp.exp(-logits))
    rem = jnp.ones_like(lam[0:1])
    ps = []
    for n in range(NITER):
        ps.append(lam[n:n + 1] * rem)
        rem = rem * (1.0 - lam[n:n + 1])
    ps[-1] = ps[-1] + rem
    p = jnp.concatenate(ps, axis=0)
    lo_ref[...] = logits
    p_ref[...] = p


def _halting(outs_pad, Wh, bh):
    blk = 1280
    grid = NPAD // blk
    lo, p = pl.pallas_call(
        _halt_body,
        grid=(grid,),
        in_specs=[
            pl.BlockSpec((NITER, blk, C), lambda i: (0, i, 0)),
            pl.BlockSpec((1, C), lambda i: (0, 0)),
            pl.BlockSpec((1, 1), lambda i: (0, 0)),
        ],
        out_specs=[
            pl.BlockSpec((NITER, blk), lambda i: (0, i)),
            pl.BlockSpec((NITER, blk), lambda i: (0, i)),
        ],
        out_shape=[
            jax.ShapeDtypeStruct((NITER, NPAD), jnp.float32),
            jax.ShapeDtypeStruct((NITER, NPAD), jnp.float32),
        ],
    )(outs_pad, Wh, bh.reshape(1, 1))
    return lo[:, :N].T, p[:, :N].T


@jax.jit
def kernel(x, edge_index, W1, b1, W2, b2, Wh, bh):
    h = _mlp(x, W1, b1, W2, b2)

    eidx_pad = jnp.pad(edge_index, ((0, 0), (0, EPAD - E)), constant_values=N)
    eidx_pad = eidx_pad.reshape(2, NSUB * NSUP, CHW)
    h_pad = jnp.pad(h, ((0, NPAD - N), (0, 0)))
    h_split = h_pad.reshape(NPAD, NCORE, HALF).transpose(1, 0, 2)
    h_split = h_split.reshape(NCORE * NPAD, HALF)

    outs_flat, _u = _propagate(eidx_pad, h_split)

    outs_pad = outs_flat.reshape(NITER, NCORE, NPAD, HALF)
    outs_pad = outs_pad.transpose(0, 2, 1, 3).reshape(NITER, NPAD, C)
    outs = outs_pad[:, :N]

    logits, p = _halting(outs_pad, Wh, bh)
    stacked = jnp.concatenate([h[None], outs], axis=0)
    return (stacked, p, logits)


# speedup vs baseline: 1.7554x; 1.7554x over previous
import functools

import jax
import jax.numpy as jnp
from jax import lax
from jax.experimental import pallas as pl
from jax.experimental.pallas import tpu as pltpu
from jax.experimental.pallas import tpu_sc as plsc

N = 10000
E = 320000
D = 128
H = 64
C = 64
NITER = 10

NCORE = 2
NSUB = 16
HALF = C // NCORE
NPAD = 10240
RPT = NPAD // NSUB
NRSUB = RPT // 128
EPT = 20480
EPAD = EPT * NSUB
NCHUNK = EPT // 128


def _mlp_body(x_ref, w1_ref, b1_ref, w2_ref, b2_ref, o_ref):
    x = x_ref[...]
    h1 = lax.dot_general(x, w1_ref[...], (((1,), (1,)), ((), ())),
                         preferred_element_type=jnp.float32)
    h1 = jnp.maximum(h1 + b1_ref[...], 0.0)
    h2 = lax.dot_general(h1, w2_ref[...], (((1,), (1,)), ((), ())),
                         preferred_element_type=jnp.float32)
    o_ref[...] = h2 + b2_ref[...]


def _mlp(x, W1, b1, W2, b2):
    blk = 1000
    grid = N // blk
    return pl.pallas_call(
        _mlp_body,
        grid=(grid,),
        in_specs=[
            pl.BlockSpec((blk, D), lambda i: (i, 0)),
            pl.BlockSpec((H, D), lambda i: (0, 0)),
            pl.BlockSpec((1, H), lambda i: (0, 0)),
            pl.BlockSpec((C, H), lambda i: (0, 0)),
            pl.BlockSpec((1, C), lambda i: (0, 0)),
        ],
        out_specs=pl.BlockSpec((blk, C), lambda i: (i, 0)),
        out_shape=jax.ShapeDtypeStruct((N, C), jnp.float32),
    )(x, W1, b1.reshape(1, H), W2, b2.reshape(1, C))


def _rsqrt16(x):
    s = (x + 1.0) * 0.5
    for _ in range(8):
        s = 0.5 * (s + x / s)
    return 1.0 / s


CHW = 256
NSUP = EPT // CHW


def _prop_body(eidx, h_hbm, outs_hbm,
               src_v, dst_v, gbuf, ones_v, degbuf,
               dinv1_v, deg_sh, acc_sh, u_sh, semg, sems):
    core = lax.axis_index("c")
    wid = lax.axis_index("s")
    hbuf = gbuf.at[pl.ds(0, 128)]
    ubuf = gbuf.at[pl.ds(128, 128)]
    accbuf = gbuf.at[pl.ds(256, 128)]
    rbase = wid * RPT
    ubase = core * NPAD

    pltpu.sync_copy(eidx.at[0, pl.ds(wid * NSUP, NSUP)], src_v)
    pltpu.sync_copy(eidx.at[1, pl.ds(wid * NSUP, NSUP)], dst_v)

    def _ones_body(k, _):
        ones_v[k, :] = jnp.full((16,), 1.0, jnp.float32)
        return 0
    lax.fori_loop(0, CHW, _ones_body, 0)

    def _dinit_body(s, _):
        pltpu.sync_copy(ones_v.at[pl.ds(0, 128)],
                        deg_sh.at[pl.ds(rbase + s * 128, 128)])
        return 0
    lax.fori_loop(0, NRSUB, _dinit_body, 0)
    plsc.subcore_barrier()

    def _deg_body(j, _):
        pltpu.sync_copy(ones_v, deg_sh.at[dst_v.at[j]], add=True)
        return 0
    lax.fori_loop(0, NSUP, _deg_body, 0)
    plsc.subcore_barrier()

    def _dinv_sub(s, _):
        pltpu.sync_copy(deg_sh.at[pl.ds(rbase + s * 128, 128)], degbuf)

        def _row(i, _):
            dinv1_v[s * 128 + i, :] = _rsqrt16(degbuf[i, :])
            return 0
        lax.fori_loop(0, 128, _row, 0)
        return 0
    lax.fori_loop(0, NRSUB, _dinv_sub, 0)

    def _u0_sub(s, _):
        pltpu.sync_copy(h_hbm.at[pl.ds(ubase + rbase + s * 128, 128)], hbuf)

        def _row(i, _):
            dv = dinv1_v[s * 128 + i, :]
            for p in range(HALF // 16):
                ubuf[i, pl.ds(p * 16, 16)] = hbuf[i, pl.ds(p * 16, 16)] * dv
            return 0
        lax.fori_loop(0, 128, _row, 0)
        pltpu.sync_copy(ubuf, u_sh.at[pl.ds(rbase + s * 128, 128)])
        pltpu.sync_copy(ubuf, acc_sh.at[pl.ds(rbase + s * 128, 128)])
        return 0
    lax.fori_loop(0, NRSUB, _u0_sub, 0)
    plsc.subcore_barrier()

    def _round(t, _):
        pltpu.async_copy(u_sh.at[src_v.at[0]],
                         gbuf.at[pl.ds(0, CHW)], semg.at[0])

        def _grp(g, _):
            for b in range(2):
                s = g * 2 + b
                bb = (b + 1) % 2
                slot = pl.ds(b * CHW, CHW)
                oth = pl.ds(bb * CHW, CHW)
                pltpu.make_async_copy(
                    u_sh.at[src_v.at[s]], gbuf.at[slot], semg.at[b]).wait()
                pltpu.async_copy(gbuf.at[slot], acc_sh.at[dst_v.at[s]],
                                 sems.at[b], add=True)

                @pl.when(s + 1 < NSUP)
                def _():
                    @pl.when(s >= 1)
                    def _():
                        pltpu.make_async_copy(
                            gbuf.at[oth], acc_sh.at[dst_v.at[s - 1]],
                            sems.at[bb]).wait()
                    pltpu.async_copy(u_sh.at[src_v.at[s + 1]],
                                     gbuf.at[oth], semg.at[bb])
            return 0
        lax.fori_loop(0, NSUP // 2, _grp, 0)
        for b in range(2):
            sl = NSUP - 2 + b
            pltpu.make_async_copy(gbuf.at[pl.ds(b * CHW, CHW)],
                                  acc_sh.at[dst_v.at[sl]], sems.at[b]).wait()
        plsc.subcore_barrier()

        def _scale_sub(s, _):
            row0 = rbase + s * 128
            pltpu.sync_copy(acc_sh.at[pl.ds(row0, 128)], accbuf)

            def _row(i, _):
                d1 = dinv1_v[s * 128 + i, :]
                d2 = d1 * d1
                for p in range(HALF // 16):
                    a = accbuf[i, pl.ds(p * 16, 16)]
                    hbuf[i, pl.ds(p * 16, 16)] = a * d1
                    ubuf[i, pl.ds(p * 16, 16)] = a * d2
                return 0
            lax.fori_loop(0, 128, _row, 0)
            orow = t * (NCORE * NPAD) + ubase + row0
            pltpu.sync_copy(hbuf, outs_hbm.at[pl.ds(orow, 128)])
            pltpu.sync_copy(ubuf, u_sh.at[pl.ds(row0, 128)])
            pltpu.sync_copy(ubuf, acc_sh.at[pl.ds(row0, 128)])
            return 0
        lax.fori_loop(0, NRSUB, _scale_sub, 0)
        plsc.subcore_barrier()
        return 0
    lax.fori_loop(0, NITER, _round, 0)


def _propagate(eidx_pad, h_split):
    mesh = plsc.VectorSubcoreMesh(core_axis_name="c", subcore_axis_name="s",
                                  num_cores=NCORE, num_subcores=NSUB)
    f = pl.kernel(
        _prop_body,
        out_type=jax.ShapeDtypeStruct((NITER * NCORE * NPAD, HALF),
                                      jnp.float32),
        mesh=mesh,
        scratch_types=[
            pltpu.VMEM((NSUP, CHW), jnp.int32),
            pltpu.VMEM((NSUP, CHW), jnp.int32),
            pltpu.VMEM((2 * CHW, HALF), jnp.float32),
            pltpu.VMEM((CHW, 16), jnp.float32),
            pltpu.VMEM((128, 16), jnp.float32),
            pltpu.VMEM((RPT, 16), jnp.float32),
            pltpu.VMEM_SHARED((NPAD, 16), jnp.float32),
            pltpu.VMEM_SHARED((NPAD, HALF), jnp.float32),
            pltpu.VMEM_SHARED((NPAD, HALF), jnp.float32),
            pltpu.SemaphoreType.DMA((2,)),
            pltpu.SemaphoreType.DMA((2,)),
        ],
        compiler_params=pltpu.CompilerParams(use_tc_tiling_on_sc=False),
    )
    return f(eidx_pad, h_split)


def _halt_body(o_ref, wh_ref, bh_ref, lo_ref, p_ref):
    wh = wh_ref[...]
    rows = []
    for t in range(NITER):
        lt = jnp.sum(o_ref[t] * wh, axis=1)[None, :] + bh_ref[...]
        rows.append(jnp.clip(lt, -10.0, 10.0))
    logits = jnp.concatenate(rows, axis=0)
    lam = 1.0 / (1.0 + jnp.exp(-logits))
    rem = jnp.ones_like(lam[0:1])
    ps = []
    for n in range(NITER):
        ps.append(lam[n:n + 1] * rem)
        rem = rem * (1.0 - lam[n:n + 1])
    ps[-1] = ps[-1] + rem
    p = jnp.concatenate(ps, axis=0)
    lo_ref[...] = logits
    p_ref[...] = p


def _halting(outs_pad, Wh, bh):
    blk = 1280
    grid = NPAD // blk
    lo, p = pl.pallas_call(
        _halt_body,
        grid=(grid,),
        in_specs=[
            pl.BlockSpec((NITER, blk, C), lambda i: (0, i, 0)),
            pl.BlockSpec((1, C), lambda i: (0, 0)),
            pl.BlockSpec((1, 1), lambda i: (0, 0)),
        ],
        out_specs=[
            pl.BlockSpec((NITER, blk), lambda i: (0, i)),
            pl.BlockSpec((NITER, blk), lambda i: (0, i)),
        ],
        out_shape=[
            jax.ShapeDtypeStruct((NITER, NPAD), jnp.float32),
            jax.ShapeDtypeStruct((NITER, NPAD), jnp.float32),
        ],
    )(outs_pad, Wh, bh.reshape(1, 1))
    return lo[:, :N].T, p[:, :N].T


@jax.jit
def kernel(x, edge_index, W1, b1, W2, b2, Wh, bh):
    h = _mlp(x, W1, b1, W2, b2)

    eidx_pad = jnp.pad(edge_index, ((0, 0), (0, EPAD - E)), constant_values=N)
    eidx_pad = eidx_pad.reshape(2, NSUB * NSUP, CHW)
    h_pad = jnp.pad(h, ((0, NPAD - N), (0, 0)))
    h_split = h_pad.reshape(NPAD, NCORE, HALF).transpose(1, 0, 2)
    h_split = h_split.reshape(NCORE * NPAD, HALF)

    outs_flat = _propagate(eidx_pad, h_split)

    outs_pad = outs_flat.reshape(NITER, NCORE, NPAD, HALF)
    outs_pad = outs_pad.transpose(0, 2, 1, 3).reshape(NITER, NPAD, C)
    outs = outs_pad[:, :N]

    logits, p = _halting(outs_pad, Wh, bh)
    stacked = jnp.concatenate([h[None], outs], axis=0)
    return (stacked, p, logits)


# speedup vs baseline: 1.9089x; 1.0875x over previous
import functools

import jax
import jax.numpy as jnp
from jax import lax
from jax.experimental import pallas as pl
from jax.experimental.pallas import tpu as pltpu
from jax.experimental.pallas import tpu_sc as plsc

N = 10000
E = 320000
D = 128
H = 64
C = 64
NITER = 10

NCORE = 2
NSUB = 16
HALF = C // NCORE
NPAD = 10240
RPT = NPAD // NSUB
NRSUB = RPT // 128
EPT = 20480
EPAD = EPT * NSUB
NCHUNK = EPT // 128


def _mlp_body(x_ref, w1_ref, b1_ref, w2_ref, b2_ref, o_ref):
    x = x_ref[...]
    h1 = lax.dot_general(x, w1_ref[...], (((1,), (1,)), ((), ())),
                         preferred_element_type=jnp.float32)
    h1 = jnp.maximum(h1 + b1_ref[...], 0.0)
    h2 = lax.dot_general(h1, w2_ref[...], (((1,), (1,)), ((), ())),
                         preferred_element_type=jnp.float32)
    o_ref[...] = h2 + b2_ref[...]


def _mlp(x, W1, b1, W2, b2):
    blk = 1000
    grid = N // blk
    return pl.pallas_call(
        _mlp_body,
        grid=(grid,),
        in_specs=[
            pl.BlockSpec((blk, D), lambda i: (i, 0)),
            pl.BlockSpec((H, D), lambda i: (0, 0)),
            pl.BlockSpec((1, H), lambda i: (0, 0)),
            pl.BlockSpec((C, H), lambda i: (0, 0)),
            pl.BlockSpec((1, C), lambda i: (0, 0)),
        ],
        out_specs=pl.BlockSpec((blk, C), lambda i: (i, 0)),
        out_shape=jax.ShapeDtypeStruct((N, C), jnp.float32),
    )(x, W1, b1.reshape(1, H), W2, b2.reshape(1, C))


def _rsqrt16(x):
    s = (x + 1.0) * 0.5
    for _ in range(8):
        s = 0.5 * (s + x / s)
    return 1.0 / s


CHW = 256
NSUP = EPT // CHW


def _prop_body(eidx, h_hbm, outs_hbm,
               src_v, dst_v, gbuf, ones_v, degbuf,
               dinv1_v, acc_sh, u_sh, semg, sems):
    core = lax.axis_index("c")
    wid = lax.axis_index("s")
    hbuf = gbuf.at[pl.ds(0, 128)]
    ubuf = gbuf.at[pl.ds(128, 128)]
    accbuf = gbuf.at[pl.ds(256, 128)]
    rbase = wid * RPT
    ubase = core * NPAD

    pltpu.sync_copy(eidx.at[0, pl.ds(wid * NSUP, NSUP)], src_v)
    pltpu.sync_copy(eidx.at[1, pl.ds(wid * NSUP, NSUP)], dst_v)

    def _ones_body(k, _):
        ones_v[k // 2, pl.ds((k % 2) * 16, 16)] = jnp.full((16,), 1.0,
                                                          jnp.float32)
        return 0
    lax.fori_loop(0, CHW * 2, _ones_body, 0)

    def _dinit_body(s, _):
        pltpu.sync_copy(ones_v.at[pl.ds(0, 128)],
                        acc_sh.at[pl.ds(rbase + s * 128, 128)])
        return 0
    lax.fori_loop(0, NRSUB, _dinit_body, 0)
    plsc.subcore_barrier()

    def _deg_body(j, _):
        pltpu.sync_copy(ones_v, acc_sh.at[dst_v.at[j]], add=True)
        return 0
    lax.fori_loop(0, NSUP, _deg_body, 0)
    plsc.subcore_barrier()

    def _dinv_sub(s, _):
        pltpu.sync_copy(acc_sh.at[pl.ds(rbase + s * 128, 128)], degbuf)

        def _row(i, _):
            dinv1_v[s * 128 + i, :] = _rsqrt16(degbuf[i, pl.ds(0, 16)])
            return 0
        lax.fori_loop(0, 128, _row, 0)
        return 0
    lax.fori_loop(0, NRSUB, _dinv_sub, 0)

    def _u0_sub(s, _):
        pltpu.sync_copy(h_hbm.at[pl.ds(ubase + rbase + s * 128, 128)], hbuf)

        def _row(i, _):
            dv = dinv1_v[s * 128 + i, :]
            for p in range(HALF // 16):
                ubuf[i, pl.ds(p * 16, 16)] = hbuf[i, pl.ds(p * 16, 16)] * dv
            return 0
        lax.fori_loop(0, 128, _row, 0)
        pltpu.sync_copy(ubuf, u_sh.at[pl.ds(rbase + s * 128, 128)])
        pltpu.sync_copy(ubuf, acc_sh.at[pl.ds(rbase + s * 128, 128)])
        return 0
    lax.fori_loop(0, NRSUB, _u0_sub, 0)
    plsc.subcore_barrier()

    NB = 3
    PF = 2

    def _round(t, _):
        for s0 in range(PF):
            pltpu.async_copy(u_sh.at[src_v.at[s0]],
                             gbuf.at[pl.ds(s0 * CHW, CHW)], semg.at[s0])

        def _step(s, _):
            b = s % NB
            slot = pl.ds(b * CHW, CHW)
            pltpu.make_async_copy(
                u_sh.at[src_v.at[s]], gbuf.at[slot], semg.at[b]).wait()
            pltpu.async_copy(gbuf.at[slot], acc_sh.at[dst_v.at[s]],
                             sems.at[b], add=True)
            jj = s + PF
            bb = jj % NB
            slot2 = pl.ds(bb * CHW, CHW)

            @pl.when(jj < NSUP)
            def _():
                @pl.when(jj >= NB)
                def _():
                    pltpu.make_async_copy(
                        gbuf.at[slot2], acc_sh.at[dst_v.at[jj - NB]],
                        sems.at[bb]).wait()
                pltpu.async_copy(u_sh.at[src_v.at[jj]],
                                 gbuf.at[slot2], semg.at[bb])
            return 0
        lax.fori_loop(0, NSUP, _step, 0)

        def _drain(k, _):
            sl = NSUP - NB + k
            b = sl % NB
            pltpu.make_async_copy(gbuf.at[pl.ds(b * CHW, CHW)],
                                  acc_sh.at[dst_v.at[sl]], sems.at[b]).wait()
            return 0
        lax.fori_loop(0, NB, _drain, 0)
        plsc.subcore_barrier()

        def _scale_sub(s, _):
            row0 = rbase + s * 128
            pltpu.sync_copy(acc_sh.at[pl.ds(row0, 128)], accbuf)

            def _row(i, _):
                d1 = dinv1_v[s * 128 + i, :]
                d2 = d1 * d1
                for p in range(HALF // 16):
                    a = accbuf[i, pl.ds(p * 16, 16)]
                    hbuf[i, pl.ds(p * 16, 16)] = a * d1
                    ubuf[i, pl.ds(p * 16, 16)] = a * d2
                return 0
            lax.fori_loop(0, 128, _row, 0)
            orow = t * (NCORE * NPAD) + ubase + row0
            pltpu.sync_copy(hbuf, outs_hbm.at[pl.ds(orow, 128)])
            pltpu.sync_copy(ubuf, u_sh.at[pl.ds(row0, 128)])
            pltpu.sync_copy(ubuf, acc_sh.at[pl.ds(row0, 128)])
            return 0
        lax.fori_loop(0, NRSUB, _scale_sub, 0)
        plsc.subcore_barrier()
        return 0
    lax.fori_loop(0, NITER, _round, 0)


def _propagate(eidx_pad, h_split):
    mesh = plsc.VectorSubcoreMesh(core_axis_name="c", subcore_axis_name="s",
                                  num_cores=NCORE, num_subcores=NSUB)
    f = pl.kernel(
        _prop_body,
        out_type=jax.ShapeDtypeStruct((NITER * NCORE * NPAD, HALF),
                                      jnp.float32),
        mesh=mesh,
        scratch_types=[
            pltpu.VMEM((NSUP, CHW), jnp.int32),
            pltpu.VMEM((NSUP, CHW), jnp.int32),
            pltpu.VMEM((3 * CHW, HALF), jnp.float32),
            pltpu.VMEM((CHW, HALF), jnp.float32),
            pltpu.VMEM((128, HALF), jnp.float32),
            pltpu.VMEM((RPT, 16), jnp.float32),
            pltpu.VMEM_SHARED((NPAD, HALF), jnp.float32),
            pltpu.VMEM_SHARED((NPAD, HALF), jnp.float32),
            pltpu.SemaphoreType.DMA((3,)),
            pltpu.SemaphoreType.DMA((3,)),
        ],
        compiler_params=pltpu.CompilerParams(use_tc_tiling_on_sc=False),
    )
    return f(eidx_pad, h_split)


def _halt_body(o_ref, wh_ref, bh_ref, lo_ref, p_ref):
    wh = wh_ref[...]
    rows = []
    for t in range(NITER):
        lt = jnp.sum(o_ref[t] * wh, axis=1)[None, :] + bh_ref[...]
        rows.append(jnp.clip(lt, -10.0, 10.0))
    logits = jnp.concatenate(rows, axis=0)
    lam = 1.0 / (1.0 + jnp.exp(-logits))
    rem = jnp.ones_like(lam[0:1])
    ps = []
    for n in range(NITER):
        ps.append(lam[n:n + 1] * rem)
        rem = rem * (1.0 - lam[n:n + 1])
    ps[-1] = ps[-1] + rem
    p = jnp.concatenate(ps, axis=0)
    lo_ref[...] = logits
    p_ref[...] = p


def _halting(outs_pad, Wh, bh):
    blk = 1280
    grid = NPAD // blk
    lo, p = pl.pallas_call(
        _halt_body,
        grid=(grid,),
        in_specs=[
            pl.BlockSpec((NITER, blk, C), lambda i: (0, i, 0)),
            pl.BlockSpec((1, C), lambda i: (0, 0)),
            pl.BlockSpec((1, 1), lambda i: (0, 0)),
        ],
        out_specs=[
            pl.BlockSpec((NITER, blk), lambda i: (0, i)),
            pl.BlockSpec((NITER, blk), lambda i: (0, i)),
        ],
        out_shape=[
            jax.ShapeDtypeStruct((NITER, NPAD), jnp.float32),
            jax.ShapeDtypeStruct((NITER, NPAD), jnp.float32),
        ],
    )(outs_pad, Wh, bh.reshape(1, 1))
    return lo[:, :N].T, p[:, :N].T


@jax.jit
def kernel(x, edge_index, W1, b1, W2, b2, Wh, bh):
    h = _mlp(x, W1, b1, W2, b2)

    eidx_pad = jnp.pad(edge_index, ((0, 0), (0, EPAD - E)), constant_values=N)
    eidx_pad = eidx_pad.reshape(2, NSUB * NSUP, CHW)
    h_pad = jnp.pad(h, ((0, NPAD - N), (0, 0)))
    h_split = h_pad.reshape(NPAD, NCORE, HALF).transpose(1, 0, 2)
    h_split = h_split.reshape(NCORE * NPAD, HALF)

    outs_flat = _propagate(eidx_pad, h_split)

    outs_pad = outs_flat.reshape(NITER, NCORE, NPAD, HALF)
    outs_pad = outs_pad.transpose(0, 2, 1, 3).reshape(NITER, NPAD, C)
    outs = outs_pad[:, :N]

    logits, p = _halting(outs_pad, Wh, bh)
    stacked = jnp.concatenate([h[None], outs], axis=0)
    return (stacked, p, logits)
